# baseline (speedup 1.0000x reference)
import functools

import jax
import jax.numpy as jnp
from jax import lax
from jax.experimental import pallas as pl
from jax.experimental.pallas import tpu as pltpu
from jax.experimental.pallas import tpu_sc as plsc

NBINS = 64
COUT = 32
K = 3
H = 384
HO = H - K + 1
NMAPS = 16 * COUT
TILES_PER_MAP = (H // 8) * (H // 128)
NTILES = NMAPS * TILES_PER_MAP
NW = 32
MAPS_PER_W = NMAPS // NW
W_TILES = MAPS_PER_W * TILES_PER_MAP
CHUNK_T = 32
NCHUNK = W_TILES // CHUNK_T
TRASH = MAPS_PER_W * NBINS
HIST_SIZE = (TRASH + 1) * 16


def _conv_idx_kernel(x_ref, w_ref, b_ref, out_ref, scr_ref):
    c = pl.program_id(1)

    @pl.when(c == 0)
    def _():
        for di in range(K):
            for dj in range(K):
                scr_ref[di * K + dj, :, 0:H - dj] = x_ref[0, di:di + HO, dj:H]

    acc = jnp.zeros((HO, H), dtype=jnp.float32)
    for k in range(K * K):
        acc = acc + w_ref[c, k] * scr_ref[k]
    y = jnp.maximum(acc + b_ref[c], 0.0)
    colmask = lax.broadcasted_iota(jnp.int32, (HO, H), 1) < HO
    lo = jnp.min(jnp.where(colmask, y, jnp.inf))
    hi = jnp.max(jnp.where(colmask, y, -jnp.inf))
    same = hi == lo
    lo = jnp.where(same, lo - 1.0, lo)
    hi = jnp.where(same, hi + 1.0, hi)
    scale = NBINS / (hi - lo)
    idx = jnp.floor((y - lo) * scale).astype(jnp.int32)
    idx = jnp.clip(idx, 0, NBINS - 1)
    idx = idx + lax.rem(c, MAPS_PER_W) * NBINS
    idx = jnp.where(colmask, idx, TRASH)
    idx384 = jnp.pad(idx, ((0, H - HO), (0, 0)), constant_values=TRASH)
    out_ref[...] = (
        idx384.reshape(H // 8, 8, H // 128, 128)
        .transpose(0, 2, 1, 3)
        .reshape(TILES_PER_MAP, 8, 128))


def _sc_hist(idx_hbm, out_hbm, buf0, buf1, hist, merged, sem0, sem1):
    wid = lax.axis_index("s") * 2 + lax.axis_index("c")
    base = wid * W_TILES
    lanes = lax.iota(jnp.int32, 16)
    ones = jnp.ones((16,), jnp.float32)
    zeros = jnp.zeros((16,), jnp.float32)

    def zero_body(i, _):
        hist[pl.ds(i * 16, 16)] = zeros
        return 0

    lax.fori_loop(0, HIST_SIZE // 16, zero_body, 0)

    def process(buf):
        @plsc.parallel_loop(0, CHUNK_T, unroll=4)
        def _(t):
            for s in range(8):
                for l in range(8):
                    v = buf[t, s, pl.ds(l * 16, 16)]
                    plsc.addupdate_scatter(
                        hist, [lax.shift_left(v, 4) + lanes], ones)

    pltpu.async_copy(idx_hbm.at[pl.ds(base, CHUNK_T)], buf0, sem0)

    def pair_body(p, _):
        off = base + 2 * p * CHUNK_T
        c1 = pltpu.async_copy(
            idx_hbm.at[pl.ds(off + CHUNK_T, CHUNK_T)], buf1, sem1)
        pltpu.make_async_copy(
            idx_hbm.at[pl.ds(off, CHUNK_T)], buf0, sem0).wait()
        process(buf0)

        @pl.when(p < NCHUNK // 2 - 1)
        def _():
            pltpu.async_copy(
                idx_hbm.at[pl.ds(off + 2 * CHUNK_T, CHUNK_T)], buf0, sem0)

        c1.wait()
        process(buf1)
        return 0

    lax.fori_loop(0, NCHUNK // 2, pair_body, 0)

    bins16 = lanes * 16

    def merge_body(j, _):
        acc = zeros
        for l in range(16):
            acc = acc + plsc.load_gather(hist, [bins16 + (j * 256 + l)])
        merged[pl.ds(j * 16, 16)] = acc
        return 0

    lax.fori_loop(0, TRASH // 16, merge_body, 0)
    pltpu.sync_copy(merged, out_hbm.at[pl.ds(wid * TRASH, TRASH)])


def _head_kernel(h_ref, w_ref, b_ref, o_ref):
    o_ref[...] = (
        jnp.dot(h_ref[...], w_ref[...], preferred_element_type=jnp.float32)
        + b_ref[...].reshape(1, -1))


def kernel(x, conv_w, conv_b, head_w, head_b):
    B = x.shape[0]
    FC = head_w.shape[0]
    xs = x.reshape(B, H, H)
    wf = conv_w.reshape(COUT, K * K)

    idx_tiles = pl.pallas_call(
        _conv_idx_kernel,
        grid=(B, COUT),
        in_specs=[
            pl.BlockSpec((1, H, H), lambda b, c: (b, 0, 0)),
            pl.BlockSpec(memory_space=pltpu.SMEM),
            pl.BlockSpec(memory_space=pltpu.SMEM),
        ],
        out_specs=pl.BlockSpec(
            (TILES_PER_MAP, 8, 128),
            lambda b, c: (b * COUT + c, 0, 0)),
        out_shape=jax.ShapeDtypeStruct((NTILES, 8, 128), jnp.int32),
        scratch_shapes=[pltpu.VMEM((K * K, HO, H), jnp.float32)],
    )(xs, wf, conv_b)

    sc_hist = functools.partial(
        pl.kernel,
        mesh=plsc.VectorSubcoreMesh(core_axis_name="c", subcore_axis_name="s"),
        compiler_params=pltpu.CompilerParams(needs_layout_passes=False),
        out_type=jax.ShapeDtypeStruct((NMAPS * NBINS,), jnp.float32),
        scratch_types=[
            pltpu.VMEM((CHUNK_T, 8, 128), jnp.int32),
            pltpu.VMEM((CHUNK_T, 8, 128), jnp.int32),
            pltpu.VMEM((HIST_SIZE,), jnp.float32),
            pltpu.VMEM((TRASH,), jnp.float32),
            pltpu.SemaphoreType.DMA,
            pltpu.SemaphoreType.DMA,
        ],
    )(_sc_hist)

    counts = sc_hist(idx_tiles)
    h = counts.reshape(B, COUT * NBINS)

    out = pl.pallas_call(
        _head_kernel,
        in_specs=[
            pl.BlockSpec((B, COUT * NBINS), lambda: (0, 0)),
            pl.BlockSpec((COUT * NBINS, FC), lambda: (0, 0)),
            pl.BlockSpec((FC,), lambda: (0,)),
        ],
        out_specs=pl.BlockSpec((B, FC), lambda: (0, 0)),
        out_shape=jax.ShapeDtypeStruct((B, FC), jnp.float32),
    )(h, head_w.T, head_b)
    return out


# speedup vs baseline: 1.2836x; 1.2836x over previous
import functools

import jax
import jax.numpy as jnp
from jax import lax
from jax.experimental import pallas as pl
from jax.experimental.pallas import tpu as pltpu
from jax.experimental.pallas import tpu_sc as plsc

NBINS = 64
COUT = 32
K = 3
H = 384
HO = H - K + 1
B = 16
G = 4
BG = B // G
TILES_PER_MAP = (H // 8) * (H // 128)
GMAPS = BG * COUT
GTILES = GMAPS * TILES_PER_MAP
NW = 32
MAPS_PER_W = GMAPS // NW
W_TILES = MAPS_PER_W * TILES_PER_MAP
CHUNK_T = 32
NCHUNK = W_TILES // CHUNK_T
TRASH = MAPS_PER_W * NBINS
HIST_SIZE = (TRASH + 1) * 16


def _conv_idx_kernel(x_ref, w_ref, b_ref, out_ref, scr_ref):
    c = pl.program_id(1)

    @pl.when(c == 0)
    def _():
        for di in range(K):
            for dj in range(K):
                scr_ref[di * K + dj, :, 0:H - dj] = x_ref[0, di:di + HO, dj:H]

    acc = jnp.zeros((HO, H), dtype=jnp.float32)
    for k in range(K * K):
        acc = acc + w_ref[c, k] * scr_ref[k]
    y = jnp.maximum(acc + b_ref[c], 0.0)
    colmask = lax.broadcasted_iota(jnp.int32, (HO, H), 1) < HO
    lo = jnp.min(jnp.where(colmask, y, jnp.inf))
    hi = jnp.max(jnp.where(colmask, y, -jnp.inf))
    same = hi == lo
    lo = jnp.where(same, lo - 1.0, lo)
    hi = jnp.where(same, hi + 1.0, hi)
    scale = NBINS / (hi - lo)
    idx = jnp.floor((y - lo) * scale).astype(jnp.int32)
    idx = jnp.clip(idx, 0, NBINS - 1)
    idx = idx + lax.rem(c, MAPS_PER_W) * NBINS
    idx = jnp.where(colmask, idx, TRASH)
    idx384 = jnp.pad(idx, ((0, H - HO), (0, 0)), constant_values=TRASH)
    out_ref[...] = (
        idx384.reshape(H // 8, 8, H // 128, 128)
        .transpose(0, 2, 1, 3)
        .reshape(TILES_PER_MAP, 8, 128))


def _sc_hist(idx_hbm, out_hbm, buf0, buf1, hist, merged, sem0, sem1):
    wid = lax.axis_index("s") * 2 + lax.axis_index("c")
    base = wid * W_TILES
    lanes = lax.iota(jnp.int32, 16)
    ones = jnp.ones((16,), jnp.float32)
    zeros = jnp.zeros((16,), jnp.float32)

    def zero_body(i, _):
        hist[pl.ds(i * 16, 16)] = zeros
        return 0

    lax.fori_loop(0, HIST_SIZE // 16, zero_body, 0)

    def process(buf):
        @plsc.parallel_loop(0, CHUNK_T, unroll=4)
        def _(t):
            for s in range(8):
                for l in range(8):
                    v = buf[t, s, pl.ds(l * 16, 16)]
                    plsc.addupdate_scatter(
                        hist, [lax.shift_left(v, 4) + lanes], ones)

    pltpu.async_copy(idx_hbm.at[pl.ds(base, CHUNK_T)], buf0, sem0)

    def pair_body(p, _):
        off = base + 2 * p * CHUNK_T
        c1 = pltpu.async_copy(
            idx_hbm.at[pl.ds(off + CHUNK_T, CHUNK_T)], buf1, sem1)
        pltpu.make_async_copy(
            idx_hbm.at[pl.ds(off, CHUNK_T)], buf0, sem0).wait()
        process(buf0)

        @pl.when(p < NCHUNK // 2 - 1)
        def _():
            pltpu.async_copy(
                idx_hbm.at[pl.ds(off + 2 * CHUNK_T, CHUNK_T)], buf0, sem0)

        c1.wait()
        process(buf1)
        return 0

    lax.fori_loop(0, NCHUNK // 2, pair_body, 0)

    bins16 = lanes * 16

    def merge_body(j, _):
        acc = zeros
        for l in range(16):
            acc = acc + plsc.load_gather(hist, [bins16 + (j * 256 + l)])
        merged[pl.ds(j * 16, 16)] = acc
        return 0

    lax.fori_loop(0, TRASH // 16, merge_body, 0)
    pltpu.sync_copy(merged, out_hbm.at[pl.ds(wid * TRASH, TRASH)])


def _head_kernel(h_ref, w_ref, b_ref, o_ref):
    o_ref[...] = (
        jnp.dot(h_ref[...], w_ref[...], preferred_element_type=jnp.float32)
        + b_ref[...].reshape(1, -1))


def kernel(x, conv_w, conv_b, head_w, head_b):
    FC = head_w.shape[0]
    xs = x.reshape(B, H, H)
    wf = conv_w.reshape(COUT, K * K)

    sc_hist = functools.partial(
        pl.kernel,
        mesh=plsc.VectorSubcoreMesh(core_axis_name="c", subcore_axis_name="s"),
        compiler_params=pltpu.CompilerParams(needs_layout_passes=False),
        out_type=jax.ShapeDtypeStruct((GMAPS * NBINS,), jnp.float32),
        scratch_types=[
            pltpu.VMEM((CHUNK_T, 8, 128), jnp.int32),
            pltpu.VMEM((CHUNK_T, 8, 128), jnp.int32),
            pltpu.VMEM((HIST_SIZE,), jnp.float32),
            pltpu.VMEM((TRASH,), jnp.float32),
            pltpu.SemaphoreType.DMA,
            pltpu.SemaphoreType.DMA,
        ],
    )(_sc_hist)

    counts = []
    for g in range(G):
        idx_tiles = pl.pallas_call(
            _conv_idx_kernel,
            grid=(BG, COUT),
            in_specs=[
                pl.BlockSpec((1, H, H), lambda b, c, g=g: (g * BG + b, 0, 0)),
                pl.BlockSpec(memory_space=pltpu.SMEM),
                pl.BlockSpec(memory_space=pltpu.SMEM),
            ],
            out_specs=pl.BlockSpec(
                (TILES_PER_MAP, 8, 128),
                lambda b, c: (b * COUT + c, 0, 0)),
            out_shape=jax.ShapeDtypeStruct((GTILES, 8, 128), jnp.int32),
            scratch_shapes=[pltpu.VMEM((K * K, HO, H), jnp.float32)],
        )(xs, wf, conv_b)
        counts.append(sc_hist(idx_tiles))

    h = jnp.concatenate(counts).reshape(B, COUT * NBINS)

    out = pl.pallas_call(
        _head_kernel,
        in_specs=[
            pl.BlockSpec((B, COUT * NBINS), lambda: (0, 0)),
            pl.BlockSpec((COUT * NBINS, FC), lambda: (0, 0)),
            pl.BlockSpec((FC,), lambda: (0,)),
        ],
        out_specs=pl.BlockSpec((B, FC), lambda: (0, 0)),
        out_shape=jax.ShapeDtypeStruct((B, FC), jnp.float32),
    )(h, head_w.T, head_b)
    return out


# speedup vs baseline: 1.6694x; 1.3006x over previous
import functools

import jax
import jax.numpy as jnp
from jax import lax
from jax.experimental import pallas as pl
from jax.experimental.pallas import tpu as pltpu
from jax.experimental.pallas import tpu_sc as plsc

NBINS = 64
COUT = 32
K = 3
H = 384
HO = H - K + 1
B = 16
G = 4
BG = B // G
TILES_PER_MAP = (H // 8) * (H // 128)
GMAPS = BG * COUT
GTILES = GMAPS * TILES_PER_MAP
NW = 32
MAPS_PER_W = GMAPS // NW
W_TILES = MAPS_PER_W * TILES_PER_MAP
CHUNK_T = 32
NCHUNK = W_TILES // CHUNK_T
TRASH = MAPS_PER_W * NBINS
HIST_SIZE = (TRASH + 1) * 16


RB = 48
NBLK = H // RB
TPB = (RB // 8) * (H // 128)


def _blk_mask(rb):
    m = lax.broadcasted_iota(jnp.int32, (RB, H), 1) < HO
    if (rb + 1) * RB > HO:
        m = m & (lax.broadcasted_iota(jnp.int32, (RB, H), 0) < HO - rb * RB)
    return m


def _conv_idx_kernel(x_ref, w_ref, b_ref, out_ref, scr_ref, y_ref):
    c = pl.program_id(1)

    @pl.when(c == 0)
    def _():
        for di in range(K):
            for dj in range(K):
                scr_ref[di * K + dj, 0:H - di, 0:H - dj] = x_ref[0, di:H, dj:H]

    bias = b_ref[c]
    lo = jnp.float32(jnp.inf)
    hi = jnp.float32(-jnp.inf)
    for rb in range(NBLK):
        acc = jnp.zeros((RB, H), dtype=jnp.float32)
        for k in range(K * K):
            acc = acc + w_ref[c, k] * scr_ref[k, rb * RB:(rb + 1) * RB, :]
        y = jnp.maximum(acc + bias, 0.0)
        m = _blk_mask(rb)
        lo = jnp.minimum(lo, jnp.min(jnp.where(m, y, jnp.inf)))
        hi = jnp.maximum(hi, jnp.max(jnp.where(m, y, -jnp.inf)))
        y_ref[rb * RB:(rb + 1) * RB, :] = y
    same = hi == lo
    lo = jnp.where(same, lo - 1.0, lo)
    hi = jnp.where(same, hi + 1.0, hi)
    scale = NBINS / (hi - lo)
    off = lax.rem(c, MAPS_PER_W) * NBINS
    for rb in range(NBLK):
        y = y_ref[rb * RB:(rb + 1) * RB, :]
        idx = jnp.floor((y - lo) * scale).astype(jnp.int32)
        idx = jnp.clip(idx, 0, NBINS - 1) + off
        idx = jnp.where(_blk_mask(rb), idx, TRASH)
        out_ref[pl.ds(rb * TPB, TPB)] = (
            idx.reshape(RB // 8, 8, H // 128, 128)
            .transpose(0, 2, 1, 3)
            .reshape(TPB, 8, 128))


def _sc_hist(idx_hbm, out_hbm, buf0, buf1, hist, merged, sem0, sem1):
    wid = lax.axis_index("s") * 2 + lax.axis_index("c")
    base = wid * W_TILES
    lanes = lax.iota(jnp.int32, 16)
    ones = jnp.ones((16,), jnp.float32)
    zeros = jnp.zeros((16,), jnp.float32)

    def zero_body(i, _):
        hist[pl.ds(i * 16, 16)] = zeros
        return 0

    lax.fori_loop(0, HIST_SIZE // 16, zero_body, 0)

    def process(buf):
        @plsc.parallel_loop(0, CHUNK_T, unroll=4)
        def _(t):
            for s in range(8):
                for l in range(8):
                    v = buf[t, s, pl.ds(l * 16, 16)]
                    plsc.addupdate_scatter(
                        hist, [lax.shift_left(v, 4) + lanes], ones)

    pltpu.async_copy(idx_hbm.at[pl.ds(base, CHUNK_T)], buf0, sem0)

    def pair_body(p, _):
        off = base + 2 * p * CHUNK_T
        c1 = pltpu.async_copy(
            idx_hbm.at[pl.ds(off + CHUNK_T, CHUNK_T)], buf1, sem1)
        pltpu.make_async_copy(
            idx_hbm.at[pl.ds(off, CHUNK_T)], buf0, sem0).wait()
        process(buf0)

        @pl.when(p < NCHUNK // 2 - 1)
        def _():
            pltpu.async_copy(
                idx_hbm.at[pl.ds(off + 2 * CHUNK_T, CHUNK_T)], buf0, sem0)

        c1.wait()
        process(buf1)
        return 0

    lax.fori_loop(0, NCHUNK // 2, pair_body, 0)

    bins16 = lanes * 16

    def merge_body(j, _):
        acc = zeros
        for l in range(16):
            acc = acc + plsc.load_gather(hist, [bins16 + (j * 256 + l)])
        merged[pl.ds(j * 16, 16)] = acc
        return 0

    lax.fori_loop(0, TRASH // 16, merge_body, 0)
    pltpu.sync_copy(merged, out_hbm.at[pl.ds(wid * TRASH, TRASH)])


def _head_kernel(h_ref, w_ref, b_ref, o_ref):
    o_ref[...] = (
        jnp.dot(h_ref[...], w_ref[...], preferred_element_type=jnp.float32)
        + b_ref[...].reshape(1, -1))


def kernel(x, conv_w, conv_b, head_w, head_b):
    FC = head_w.shape[0]
    xs = x.reshape(B, H, H)
    wf = conv_w.reshape(COUT, K * K)

    sc_hist = functools.partial(
        pl.kernel,
        mesh=plsc.VectorSubcoreMesh(core_axis_name="c", subcore_axis_name="s"),
        compiler_params=pltpu.CompilerParams(needs_layout_passes=False),
        out_type=jax.ShapeDtypeStruct((GMAPS * NBINS,), jnp.float32),
        scratch_types=[
            pltpu.VMEM((CHUNK_T, 8, 128), jnp.int32),
            pltpu.VMEM((CHUNK_T, 8, 128), jnp.int32),
            pltpu.VMEM((HIST_SIZE,), jnp.float32),
            pltpu.VMEM((TRASH,), jnp.float32),
            pltpu.SemaphoreType.DMA,
            pltpu.SemaphoreType.DMA,
        ],
    )(_sc_hist)

    counts = []
    for g in range(G):
        idx_tiles = pl.pallas_call(
            _conv_idx_kernel,
            grid=(BG, COUT),
            in_specs=[
                pl.BlockSpec((1, H, H), lambda b, c, g=g: (g * BG + b, 0, 0)),
                pl.BlockSpec(memory_space=pltpu.SMEM),
                pl.BlockSpec(memory_space=pltpu.SMEM),
            ],
            out_specs=pl.BlockSpec(
                (TILES_PER_MAP, 8, 128),
                lambda b, c: (b * COUT + c, 0, 0)),
            out_shape=jax.ShapeDtypeStruct((GTILES, 8, 128), jnp.int32),
            scratch_shapes=[
                pltpu.VMEM((K * K, H, H), jnp.float32),
                pltpu.VMEM((H, H), jnp.float32),
            ],
        )(xs, wf, conv_b)
        counts.append(sc_hist(idx_tiles))

    h = jnp.concatenate(counts).reshape(B, COUT * NBINS)

    out = pl.pallas_call(
        _head_kernel,
        in_specs=[
            pl.BlockSpec((B, COUT * NBINS), lambda: (0, 0)),
            pl.BlockSpec((COUT * NBINS, FC), lambda: (0, 0)),
            pl.BlockSpec((FC,), lambda: (0,)),
        ],
        out_specs=pl.BlockSpec((B, FC), lambda: (0, 0)),
        out_shape=jax.ShapeDtypeStruct((B, FC), jnp.float32),
    )(h, head_w.T, head_b)
    return out


# speedup vs baseline: 1.9770x; 1.1843x over previous
import functools

import jax
import jax.numpy as jnp
from jax import lax
from jax.experimental import pallas as pl
from jax.experimental.pallas import tpu as pltpu
from jax.experimental.pallas import tpu_sc as plsc

NBINS = 64
COUT = 32
K = 3
H = 384
HO = H - K + 1
B = 16
G = 4
BG = B // G
TILES_PER_MAP = (H // 8) * (H // 128)
GMAPS = BG * COUT
GTILES = GMAPS * TILES_PER_MAP
NW = 32
MAPS_PER_W = GMAPS // NW
W_TILES = MAPS_PER_W * TILES_PER_MAP
CHUNK_T = 32
NCHUNK = W_TILES // CHUNK_T
TRASH = MAPS_PER_W * NBINS
HIST_SIZE = (TRASH + 1) * 16


RB = 48
NBLK = H // RB
TPB = (RB // 8) * (H // 128)


def _blk_mask(rb):
    m = lax.broadcasted_iota(jnp.int32, (RB, H), 1) < HO
    if (rb + 1) * RB > HO:
        m = m & (lax.broadcasted_iota(jnp.int32, (RB, H), 0) < HO - rb * RB)
    return m


def _conv_idx_kernel(x_ref, w_ref, b_ref, out_ref, scr_ref, y_ref):
    c = pl.program_id(1)

    @pl.when(c == 0)
    def _():
        for di in range(K):
            for dj in range(K):
                scr_ref[di * K + dj, 0:H - di, 0:H - dj] = x_ref[0, di:H, dj:H]

    bias = b_ref[c]
    lo = jnp.float32(jnp.inf)
    hi = jnp.float32(-jnp.inf)
    for rb in range(NBLK):
        acc = jnp.zeros((RB, H), dtype=jnp.float32)
        for k in range(K * K):
            acc = acc + w_ref[c, k] * scr_ref[k, rb * RB:(rb + 1) * RB, :]
        y = jnp.maximum(acc + bias, 0.0)
        m = _blk_mask(rb)
        lo = jnp.minimum(lo, jnp.min(jnp.where(m, y, jnp.inf)))
        hi = jnp.maximum(hi, jnp.max(jnp.where(m, y, -jnp.inf)))
        y_ref[rb * RB:(rb + 1) * RB, :] = y
    same = hi == lo
    lo = jnp.where(same, lo - 1.0, lo)
    hi = jnp.where(same, hi + 1.0, hi)
    scale = NBINS / (hi - lo)
    off = lax.rem(c, MAPS_PER_W) * NBINS
    for rb in range(NBLK):
        y = y_ref[rb * RB:(rb + 1) * RB, :]
        idx = jnp.floor((y - lo) * scale).astype(jnp.int32)
        idx = jnp.clip(idx, 0, NBINS - 1) + off
        idx = jnp.where(_blk_mask(rb), idx, TRASH)
        for tc in range(H // 128):
            out_ref[pl.ds(tc * (H // 8) + rb * (RB // 8), RB // 8)] = (
                idx[:, tc * 128:(tc + 1) * 128].reshape(RB // 8, 8, 128))


def _sc_hist(idx_hbm, out_hbm, buf0, buf1, hist, merged, sem0, sem1):
    wid = lax.axis_index("s") * 2 + lax.axis_index("c")
    base = wid * W_TILES
    lanes = lax.iota(jnp.int32, 16)
    ones = jnp.ones((16,), jnp.float32)
    zeros = jnp.zeros((16,), jnp.float32)

    def zero_body(i, _):
        hist[pl.ds(i * 16, 16)] = zeros
        return 0

    lax.fori_loop(0, HIST_SIZE // 16, zero_body, 0)

    def process(buf):
        @plsc.parallel_loop(0, CHUNK_T, unroll=4)
        def _(t):
            for s in range(8):
                for l in range(8):
                    v = buf[t, s, pl.ds(l * 16, 16)]
                    plsc.addupdate_scatter(
                        hist, [lax.shift_left(v, 4) + lanes], ones)

    pltpu.async_copy(idx_hbm.at[pl.ds(base, CHUNK_T)], buf0, sem0)

    def pair_body(p, _):
        off = base + 2 * p * CHUNK_T
        c1 = pltpu.async_copy(
            idx_hbm.at[pl.ds(off + CHUNK_T, CHUNK_T)], buf1, sem1)
        pltpu.make_async_copy(
            idx_hbm.at[pl.ds(off, CHUNK_T)], buf0, sem0).wait()
        process(buf0)

        @pl.when(p < NCHUNK // 2 - 1)
        def _():
            pltpu.async_copy(
                idx_hbm.at[pl.ds(off + 2 * CHUNK_T, CHUNK_T)], buf0, sem0)

        c1.wait()
        process(buf1)
        return 0

    lax.fori_loop(0, NCHUNK // 2, pair_body, 0)

    bins16 = lanes * 16

    def merge_body(j, _):
        acc = zeros
        for l in range(16):
            acc = acc + plsc.load_gather(hist, [bins16 + (j * 256 + l)])
        merged[pl.ds(j * 16, 16)] = acc
        return 0

    lax.fori_loop(0, TRASH // 16, merge_body, 0)
    pltpu.sync_copy(merged, out_hbm.at[pl.ds(wid * TRASH, TRASH)])


def _head_kernel(h_ref, w_ref, b_ref, o_ref):
    o_ref[...] = (
        jnp.dot(h_ref[...], w_ref[...], preferred_element_type=jnp.float32)
        + b_ref[...].reshape(1, -1))


def kernel(x, conv_w, conv_b, head_w, head_b):
    FC = head_w.shape[0]
    xs = x.reshape(B, H, H)
    wf = conv_w.reshape(COUT, K * K)

    sc_hist = functools.partial(
        pl.kernel,
        mesh=plsc.VectorSubcoreMesh(core_axis_name="c", subcore_axis_name="s"),
        compiler_params=pltpu.CompilerParams(needs_layout_passes=False),
        out_type=jax.ShapeDtypeStruct((GMAPS * NBINS,), jnp.float32),
        scratch_types=[
            pltpu.VMEM((CHUNK_T, 8, 128), jnp.int32),
            pltpu.VMEM((CHUNK_T, 8, 128), jnp.int32),
            pltpu.VMEM((HIST_SIZE,), jnp.float32),
            pltpu.VMEM((TRASH,), jnp.float32),
            pltpu.SemaphoreType.DMA,
            pltpu.SemaphoreType.DMA,
        ],
    )(_sc_hist)

    counts = []
    for g in range(G):
        idx_tiles = pl.pallas_call(
            _conv_idx_kernel,
            grid=(BG, COUT),
            in_specs=[
                pl.BlockSpec((1, H, H), lambda b, c, g=g: (g * BG + b, 0, 0)),
                pl.BlockSpec(memory_space=pltpu.SMEM),
                pl.BlockSpec(memory_space=pltpu.SMEM),
            ],
            out_specs=pl.BlockSpec(
                (TILES_PER_MAP, 8, 128),
                lambda b, c: (b * COUT + c, 0, 0)),
            out_shape=jax.ShapeDtypeStruct((GTILES, 8, 128), jnp.int32),
            scratch_shapes=[
                pltpu.VMEM((K * K, H, H), jnp.float32),
                pltpu.VMEM((H, H), jnp.float32),
            ],
        )(xs, wf, conv_b)
        counts.append(sc_hist(idx_tiles))

    h = jnp.concatenate(counts).reshape(B, COUT * NBINS)

    out = pl.pallas_call(
        _head_kernel,
        in_specs=[
            pl.BlockSpec((B, COUT * NBINS), lambda: (0, 0)),
            pl.BlockSpec((COUT * NBINS, FC), lambda: (0, 0)),
            pl.BlockSpec((FC,), lambda: (0,)),
        ],
        out_specs=pl.BlockSpec((B, FC), lambda: (0, 0)),
        out_shape=jax.ShapeDtypeStruct((B, FC), jnp.float32),
    )(h, head_w.T, head_b)
    return out


# speedup vs baseline: 2.2040x; 1.1148x over previous
import functools

import jax
import jax.numpy as jnp
from jax import lax
from jax.experimental import pallas as pl
from jax.experimental.pallas import tpu as pltpu
from jax.experimental.pallas import tpu_sc as plsc

NBINS = 64
COUT = 32
K = 3
H = 384
HO = H - K + 1
B = 16
G = 8
BG = B // G
TILES_PER_MAP = (H // 8) * (H // 128)
GMAPS = BG * COUT
GTILES = GMAPS * TILES_PER_MAP
NW = 32
MAPS_PER_W = GMAPS // NW
W_WORDS = MAPS_PER_W * TILES_PER_MAP * 1024
CW = 16384
NCHUNK = W_WORDS // CW
TRASH = MAPS_PER_W * NBINS
HIST_SIZE = (TRASH + 1) * 16


RB = 48
NBLK = H // RB
TPB = (RB // 8) * (H // 128)


def _blk_mask(rb):
    m = lax.broadcasted_iota(jnp.int32, (RB, H), 1) < HO
    if (rb + 1) * RB > HO:
        m = m & (lax.broadcasted_iota(jnp.int32, (RB, H), 0) < HO - rb * RB)
    return m


def _conv_idx_kernel(x_ref, w_ref, b_ref, out_ref, scr_ref, y_ref):
    c = pl.program_id(1)

    @pl.when(c == 0)
    def _():
        for di in range(K):
            for dj in range(K):
                scr_ref[di * K + dj, 0:H - di, 0:H - dj] = x_ref[0, di:H, dj:H]

    bias = b_ref[c]
    lo = jnp.float32(jnp.inf)
    hi = jnp.float32(-jnp.inf)
    for rb in range(NBLK):
        acc = jnp.zeros((RB, H), dtype=jnp.float32)
        for k in range(K * K):
            acc = acc + w_ref[c, k] * scr_ref[k, rb * RB:(rb + 1) * RB, :]
        y = jnp.maximum(acc + bias, 0.0)
        m = _blk_mask(rb)
        lo = jnp.minimum(lo, jnp.min(jnp.where(m, y, jnp.inf)))
        hi = jnp.maximum(hi, jnp.max(jnp.where(m, y, -jnp.inf)))
        y_ref[rb * RB:(rb + 1) * RB, :] = y
    same = hi == lo
    lo = jnp.where(same, lo - 1.0, lo)
    hi = jnp.where(same, hi + 1.0, hi)
    scale = NBINS / (hi - lo)
    off = lax.rem(c, MAPS_PER_W) * NBINS
    for rb in range(NBLK):
        y = y_ref[rb * RB:(rb + 1) * RB, :]
        idx = jnp.floor((y - lo) * scale).astype(jnp.int32)
        idx = jnp.clip(idx, 0, NBINS - 1) + off
        idx = jnp.where(_blk_mask(rb), idx, TRASH)
        for tc in range(H // 128):
            out_ref[pl.ds(tc * (H // 8) + rb * (RB // 8), RB // 8)] = (
                idx[:, tc * 128:(tc + 1) * 128].reshape(RB // 8, 8, 128))


def _sc_hist(idx_hbm, out_hbm, buf0, buf1, hist, merged, sem0, sem1):
    wid = lax.axis_index("s") * 2 + lax.axis_index("c")
    base = wid * W_WORDS
    lanes = lax.iota(jnp.int32, 16)
    ones = jnp.ones((16,), jnp.float32)
    zeros = jnp.zeros((16,), jnp.float32)

    def zero_body(i, _):
        hist[pl.ds(i * 16, 16)] = zeros
        return 0

    lax.fori_loop(0, HIST_SIZE // 16, zero_body, 0)

    def process(buf):
        @plsc.parallel_loop(0, CW // 16, unroll=8)
        def _(i):
            v = buf[pl.ds(i * 16, 16)]
            plsc.addupdate_scatter(
                hist, [lax.shift_left(v, 4) + lanes], ones)

    pltpu.async_copy(idx_hbm.at[pl.ds(base, CW)], buf0, sem0)

    def pair_body(p, _):
        off = base + 2 * p * CW
        c1 = pltpu.async_copy(
            idx_hbm.at[pl.ds(off + CW, CW)], buf1, sem1)
        pltpu.make_async_copy(
            idx_hbm.at[pl.ds(off, CW)], buf0, sem0).wait()
        process(buf0)

        @pl.when(p < NCHUNK // 2 - 1)
        def _():
            pltpu.async_copy(
                idx_hbm.at[pl.ds(off + 2 * CW, CW)], buf0, sem0)

        c1.wait()
        process(buf1)
        return 0

    lax.fori_loop(0, NCHUNK // 2, pair_body, 0)

    bins16 = lanes * 16

    def merge_body(j, _):
        acc = zeros
        for l in range(16):
            acc = acc + plsc.load_gather(hist, [bins16 + (j * 256 + l)])
        merged[pl.ds(j * 16, 16)] = acc
        return 0

    lax.fori_loop(0, TRASH // 16, merge_body, 0)
    pltpu.sync_copy(merged, out_hbm.at[pl.ds(wid * TRASH, TRASH)])


def _head_kernel(h_ref, w_ref, b_ref, o_ref):
    o_ref[...] = (
        jnp.dot(h_ref[...], w_ref[...], preferred_element_type=jnp.float32)
        + b_ref[...].reshape(1, -1))


def kernel(x, conv_w, conv_b, head_w, head_b):
    FC = head_w.shape[0]
    xs = x.reshape(B, H, H)
    wf = conv_w.reshape(COUT, K * K)

    sc_hist = functools.partial(
        pl.kernel,
        mesh=plsc.VectorSubcoreMesh(core_axis_name="c", subcore_axis_name="s"),
        compiler_params=pltpu.CompilerParams(needs_layout_passes=False),
        out_type=jax.ShapeDtypeStruct((GMAPS * NBINS,), jnp.float32),
        scratch_types=[
            pltpu.VMEM((CW,), jnp.int32),
            pltpu.VMEM((CW,), jnp.int32),
            pltpu.VMEM((HIST_SIZE,), jnp.float32),
            pltpu.VMEM((TRASH,), jnp.float32),
            pltpu.SemaphoreType.DMA,
            pltpu.SemaphoreType.DMA,
        ],
    )(_sc_hist)

    counts = []
    for g in range(G):
        idx_tiles = pl.pallas_call(
            _conv_idx_kernel,
            grid=(BG, COUT),
            in_specs=[
                pl.BlockSpec((1, H, H), lambda b, c, g=g: (g * BG + b, 0, 0)),
                pl.BlockSpec(memory_space=pltpu.SMEM),
                pl.BlockSpec(memory_space=pltpu.SMEM),
            ],
            out_specs=pl.BlockSpec(
                (TILES_PER_MAP, 8, 128),
                lambda b, c: (b * COUT + c, 0, 0)),
            out_shape=jax.ShapeDtypeStruct((GTILES, 8, 128), jnp.int32),
            scratch_shapes=[
                pltpu.VMEM((K * K, H, H), jnp.float32),
                pltpu.VMEM((H, H), jnp.float32),
            ],
        )(xs, wf, conv_b)
        counts.append(sc_hist(idx_tiles.reshape(-1)))

    h = jnp.concatenate(counts).reshape(B, COUT * NBINS)

    out = pl.pallas_call(
        _head_kernel,
        in_specs=[
            pl.BlockSpec((B, COUT * NBINS), lambda: (0, 0)),
            pl.BlockSpec((COUT * NBINS, FC), lambda: (0, 0)),
            pl.BlockSpec((FC,), lambda: (0,)),
        ],
        out_specs=pl.BlockSpec((B, FC), lambda: (0, 0)),
        out_shape=jax.ShapeDtypeStruct((B, FC), jnp.float32),
    )(h, head_w.T, head_b)
    return out


# speedup vs baseline: 2.5252x; 1.1457x over previous
import functools

import jax
import jax.numpy as jnp
from jax import lax
from jax.experimental import pallas as pl
from jax.experimental.pallas import tpu as pltpu
from jax.experimental.pallas import tpu_sc as plsc

NBINS = 64
COUT = 32
K = 3
H = 384
HO = H - K + 1
B = 16
G = 8
BG = B // G
TILES_PER_MAP = (H // 8) * (H // 128)
GMAPS = BG * COUT
GTILES = GMAPS * TILES_PER_MAP
NW = 32
MAPS_PER_W = GMAPS // NW
W_WORDS = MAPS_PER_W * TILES_PER_MAP * 1024
CW = 16384
NCHUNK = W_WORDS // CW
TRASH = MAPS_PER_W * NBINS
HIST_SIZE = (TRASH + 1) * 16


RB = 48
NBLK = H // RB
TPB = (RB // 8) * (H // 128)


def _blk_mask(rb):
    m = lax.broadcasted_iota(jnp.int32, (RB, H), 1) < HO
    if (rb + 1) * RB > HO:
        m = m & (lax.broadcasted_iota(jnp.int32, (RB, H), 0) < HO - rb * RB)
    return m


CPG = 2


def _conv_idx_kernel(x_ref, w_ref, b_ref, out_ref, scr_ref, y_ref):
    cp = pl.program_id(1)
    c0 = cp * CPG

    @pl.when(cp == 0)
    def _():
        for di in range(K):
            for dj in range(K):
                scr_ref[di * K + dj, 0:H - di, 0:H - dj] = x_ref[0, di:H, dj:H]

    lo = [jnp.float32(jnp.inf)] * CPG
    hi = [jnp.float32(-jnp.inf)] * CPG
    for rb in range(NBLK):
        acc = [jnp.zeros((RB, H), dtype=jnp.float32) for _ in range(CPG)]
        for k in range(K * K):
            t = scr_ref[k, rb * RB:(rb + 1) * RB, :]
            for ch in range(CPG):
                acc[ch] = acc[ch] + w_ref[c0 + ch, k] * t
        m = _blk_mask(rb)
        for ch in range(CPG):
            y = jnp.maximum(acc[ch] + b_ref[c0 + ch], 0.0)
            lo[ch] = jnp.minimum(lo[ch], jnp.min(jnp.where(m, y, jnp.inf)))
            hi[ch] = jnp.maximum(hi[ch], jnp.max(jnp.where(m, y, -jnp.inf)))
            y_ref[ch, rb * RB:(rb + 1) * RB, :] = y
    scale = [None] * CPG
    for ch in range(CPG):
        same = hi[ch] == lo[ch]
        lo[ch] = jnp.where(same, lo[ch] - 1.0, lo[ch])
        h2 = jnp.where(same, hi[ch] + 1.0, hi[ch])
        scale[ch] = NBINS / (h2 - lo[ch])
    for rb in range(NBLK):
        m = _blk_mask(rb)
        for ch in range(CPG):
            y = y_ref[ch, rb * RB:(rb + 1) * RB, :]
            idx = jnp.floor((y - lo[ch]) * scale[ch]).astype(jnp.int32)
            idx = jnp.clip(idx, 0, NBINS - 1) + (ch % MAPS_PER_W) * NBINS
            idx = jnp.where(m, idx, TRASH)
            for tc in range(H // 128):
                out_ref[pl.ds(
                    ch * TILES_PER_MAP + tc * (H // 8) + rb * (RB // 8),
                    RB // 8)] = (
                    idx[:, tc * 128:(tc + 1) * 128].reshape(RB // 8, 8, 128))


def _sc_hist(idx_hbm, out_hbm, buf0, buf1, hist, merged, sem0, sem1):
    wid = lax.axis_index("s") * 2 + lax.axis_index("c")
    base = wid * W_WORDS
    lanes = lax.iota(jnp.int32, 16)
    ones = jnp.ones((16,), jnp.float32)
    zeros = jnp.zeros((16,), jnp.float32)

    def zero_body(i, _):
        hist[pl.ds(i * 16, 16)] = zeros
        return 0

    lax.fori_loop(0, HIST_SIZE // 16, zero_body, 0)

    def process(buf):
        @plsc.parallel_loop(0, CW // 16, unroll=8)
        def _(i):
            v = buf[pl.ds(i * 16, 16)]
            plsc.addupdate_scatter(
                hist, [lax.shift_left(v, 4) + lanes], ones)

    pltpu.async_copy(idx_hbm.at[pl.ds(base, CW)], buf0, sem0)

    def pair_body(p, _):
        off = base + 2 * p * CW
        c1 = pltpu.async_copy(
            idx_hbm.at[pl.ds(off + CW, CW)], buf1, sem1)
        pltpu.make_async_copy(
            idx_hbm.at[pl.ds(off, CW)], buf0, sem0).wait()
        process(buf0)

        @pl.when(p < NCHUNK // 2 - 1)
        def _():
            pltpu.async_copy(
                idx_hbm.at[pl.ds(off + 2 * CW, CW)], buf0, sem0)

        c1.wait()
        process(buf1)
        return 0

    lax.fori_loop(0, NCHUNK // 2, pair_body, 0)

    bins16 = lanes * 16

    def merge_body(j, _):
        acc = zeros
        for l in range(16):
            acc = acc + plsc.load_gather(hist, [bins16 + (j * 256 + l)])
        merged[pl.ds(j * 16, 16)] = acc
        return 0

    lax.fori_loop(0, TRASH // 16, merge_body, 0)
    pltpu.sync_copy(merged, out_hbm.at[pl.ds(wid * TRASH, TRASH)])


def _head_kernel(h_ref, w_ref, b_ref, o_ref):
    o_ref[...] = (
        jnp.dot(h_ref[...], w_ref[...], preferred_element_type=jnp.float32)
        + b_ref[...].reshape(1, -1))


def kernel(x, conv_w, conv_b, head_w, head_b):
    FC = head_w.shape[0]
    xs = x.reshape(B, H, H)
    wf = conv_w.reshape(COUT, K * K)

    sc_hist = functools.partial(
        pl.kernel,
        mesh=plsc.VectorSubcoreMesh(core_axis_name="c", subcore_axis_name="s"),
        compiler_params=pltpu.CompilerParams(needs_layout_passes=False),
        out_type=jax.ShapeDtypeStruct((GMAPS * NBINS,), jnp.float32),
        scratch_types=[
            pltpu.VMEM((CW,), jnp.int32),
            pltpu.VMEM((CW,), jnp.int32),
            pltpu.VMEM((HIST_SIZE,), jnp.float32),
            pltpu.VMEM((TRASH,), jnp.float32),
            pltpu.SemaphoreType.DMA,
            pltpu.SemaphoreType.DMA,
        ],
    )(_sc_hist)

    counts = []
    for g in range(G):
        idx_tiles = pl.pallas_call(
            _conv_idx_kernel,
            grid=(BG, COUT // CPG),
            in_specs=[
                pl.BlockSpec((1, H, H), lambda b, c, g=g: (g * BG + b, 0, 0)),
                pl.BlockSpec(memory_space=pltpu.SMEM),
                pl.BlockSpec(memory_space=pltpu.SMEM),
            ],
            out_specs=pl.BlockSpec(
                (CPG * TILES_PER_MAP, 8, 128),
                lambda b, c: (b * (COUT // CPG) + c, 0, 0)),
            out_shape=jax.ShapeDtypeStruct((GTILES, 8, 128), jnp.int32),
            scratch_shapes=[
                pltpu.VMEM((K * K, H, H), jnp.float32),
                pltpu.VMEM((CPG, H, H), jnp.float32),
            ],
        )(xs, wf, conv_b)
        counts.append(sc_hist(idx_tiles.reshape(-1)))

    h = jnp.concatenate(counts).reshape(B, COUT * NBINS)

    out = pl.pallas_call(
        _head_kernel,
        in_specs=[
            pl.BlockSpec((B, COUT * NBINS), lambda: (0, 0)),
            pl.BlockSpec((COUT * NBINS, FC), lambda: (0, 0)),
            pl.BlockSpec((FC,), lambda: (0,)),
        ],
        out_specs=pl.BlockSpec((B, FC), lambda: (0, 0)),
        out_shape=jax.ShapeDtypeStruct((B, FC), jnp.float32),
    )(h, head_w.T, head_b)
    return out
